# baseline (speedup 1.0000x reference)
import jax
import jax.numpy as jnp
from jax import lax
from jax.experimental import pallas as pl
from jax.experimental.pallas import tpu as pltpu
from jax.experimental.pallas import tpu_sc as plsc

N_NODE = 10000
E_TOT = 320000
DIN = 128
H = 8
F = 16
HID = 128
NACT = 32

NC = 2
NS = 16
L = 16
NW = NC * NS
EPW = E_TOT // NW
B = 80
BH = 40
NHALF = EPW // BH
N_PAD = 10240
N_TOT = N_PAD + N_PAD // 8
RPT = N_TOT // NS

F32 = jnp.float32


def _edge_body(fs, fd, si, di, d2i, dmi, attn_h, out1,
               s_sh, attn_v,
               sidx0, didx0, d2b0, dmb0,
               sidx1, didx1, d2b1, dmb1,
               el, msg, sem_i0, sem_i1, sem_g0, sem_g1, sem_c0, sem_c1):
    c = lax.axis_index("c")
    s = lax.axis_index("s")
    wid = c * NS + s

    pltpu.sync_copy(attn_h, attn_v)

    def zrow(i, _):
        for j in range(DIN // L):
            msg[i, pl.ds(j * L, L)] = jnp.zeros((L,), F32)
            el[i, pl.ds(j * L, L)] = jnp.zeros((L,), F32)
        return 0
    lax.fori_loop(0, B, zrow, 0)
    for t in range(RPT // B):
        base = s * RPT + t * B
        pltpu.sync_copy(msg, s_sh.at[pl.ds(base, B)])
    plsc.subcore_barrier()

    lane = lax.broadcasted_iota(jnp.int32, (L,), 0)
    mask8 = lane < H
    perms = [lane ^ j for j in (1, 2, 4, 8)]
    zero16 = jnp.zeros((L,), F32)
    attn_regs = [attn_v[hh] for hh in range(H)]

    def _sum16(v):
        for p in perms:
            v = v + v.at[p].get(mode="promise_in_bounds")
        return v

    slot0 = (sidx0, didx0, d2b0, dmb0, sem_i0, sem_g0, sem_c0, 0)
    slot1 = (sidx1, didx1, d2b1, dmb1, sem_i1, sem_g1, sem_c1, BH)

    def issue_idx(h, st):
        sidx, didx, d2b, dmb, sem_i, _, _, _ = st
        off = pl.multiple_of(wid * EPW + h * BH, 8)
        pltpu.async_copy(si.at[pl.ds(off, BH)], sidx, sem_i)
        pltpu.async_copy(di.at[pl.ds(off, BH)], didx, sem_i)
        pltpu.async_copy(d2i.at[pl.ds(off, BH)], d2b, sem_i)
        pltpu.async_copy(dmi.at[pl.ds(off, BH)], dmb.at[pl.ds(0, BH)], sem_i)

    def wait_idx(st):
        sidx, didx, d2b, dmb, sem_i, _, _, _ = st
        pltpu.make_async_copy(si.at[pl.ds(0, BH)], sidx, sem_i).wait()
        pltpu.make_async_copy(di.at[pl.ds(0, BH)], didx, sem_i).wait()
        pltpu.make_async_copy(d2i.at[pl.ds(0, BH)], d2b, sem_i).wait()
        pltpu.make_async_copy(dmi.at[pl.ds(0, BH)], dmb.at[pl.ds(0, BH)],
                              sem_i).wait()

    def issue_gather(st):
        sidx, didx, _, _, _, sem_g, _, base = st
        pltpu.async_copy(fs.at[sidx], el.at[pl.ds(base, BH)], sem_g)
        pltpu.async_copy(fd.at[didx], msg.at[pl.ds(base, BH)], sem_g)

    def wait_gather(st):
        _, _, _, _, _, sem_g, _, base = st
        pltpu.make_async_copy(fs.at[pl.ds(0, BH)], el.at[pl.ds(base, BH)],
                              sem_g).wait()
        pltpu.make_async_copy(fd.at[pl.ds(0, BH)], msg.at[pl.ds(base, BH)],
                              sem_g).wait()

    def issue_scatter(st):
        _, didx, d2b, _, _, _, sem_c, base = st
        pltpu.async_copy(msg.at[pl.ds(base, BH)], s_sh.at[didx], sem_c,
                         add=True)
        pltpu.async_copy(el.at[pl.ds(base, BH)], s_sh.at[d2b], sem_c,
                         add=True)

    def wait_scatter(st):
        _, _, _, _, _, _, sem_c, _ = st
        pltpu.make_async_copy(fs.at[pl.ds(0, BH)], s_sh.at[pl.ds(0, BH)],
                              sem_c).wait()
        pltpu.make_async_copy(fs.at[pl.ds(0, BH)], s_sh.at[pl.ds(0, BH)],
                              sem_c).wait()

    def compute(st):
        _, _, _, dmb, _, _, _, base = st

        def edge(e, _):
            r = base + e
            evs = []
            lgv = zero16
            for hh in range(H):
                a = el[r, pl.ds(hh * F, F)]
                b = msg[r, pl.ds(hh * F, F)]
                t = a + b
                t = jnp.maximum(t, 0.2 * t)
                lg = _sum16(t * attn_regs[hh])
                lgv = jnp.where(lane == hh, lg, lgv)
                evs.append(a)
            w = jnp.where(mask8, jnp.exp(lgv), 0.0)
            for hh in range(H):
                msg[r, pl.ds(hh * F, F)] = evs[hh] * w[hh]
            b16 = pl.multiple_of((e // L) * L, 8)
            dmv = dmb[pl.ds(b16, L)]
            dsel = dmv.at[lane * 0 + (e - b16)].get(mode="promise_in_bounds")
            for j in range(H):
                m = (1 - jnp.minimum(jnp.abs(dsel - j), 1)).astype(F32)
                el[r, pl.ds(j * F, F)] = w * m
            return 0
        lax.fori_loop(0, BH, edge, 0)

    def step(h, cur, nxt):
        wait_gather(cur)
        wait_scatter(nxt)
        issue_idx(lax.rem(h + 1, NHALF), nxt)
        compute(cur)
        wait_idx(nxt)
        issue_gather(nxt)
        issue_scatter(cur)

    off0 = pl.multiple_of(wid * EPW, 8)
    pltpu.sync_copy(si.at[pl.ds(off0, BH)], sidx0)
    pltpu.sync_copy(di.at[pl.ds(off0, BH)], didx0)
    pltpu.sync_copy(d2i.at[pl.ds(off0, BH)], d2b0)
    pltpu.sync_copy(dmi.at[pl.ds(off0, BH)], dmb0.at[pl.ds(0, BH)])
    pltpu.async_copy(msg.at[pl.ds(BH, BH)], s_sh.at[didx0], sem_c1, add=True)
    pltpu.async_copy(el.at[pl.ds(BH, BH)], s_sh.at[d2b0], sem_c1, add=True)
    issue_gather(slot0)

    def pair(kk, _):
        h0 = kk * 2
        step(h0, slot0, slot1)
        step(h0 + 1, slot1, slot0)
        return 0
    lax.fori_loop(0, NHALF // 2, pair, 0)

    wait_gather(slot0)
    wait_scatter(slot1)

    plsc.subcore_barrier()
    base = s * RPT
    pltpu.sync_copy(s_sh.at[pl.ds(base, RPT)], out1.at[c, pl.ds(base, RPT)])


_edge_call = pl.kernel(
    _edge_body,
    out_type=jax.ShapeDtypeStruct((NC, N_TOT, DIN), F32),
    mesh=plsc.VectorSubcoreMesh(core_axis_name="c", subcore_axis_name="s",
                                num_cores=NC, num_subcores=NS),
    scratch_types=[
        pltpu.VMEM_SHARED((N_TOT, DIN), F32),
        pltpu.VMEM((H, F), F32),
        pltpu.VMEM((BH,), jnp.int32),
        pltpu.VMEM((BH,), jnp.int32),
        pltpu.VMEM((BH,), jnp.int32),
        pltpu.VMEM((BH + 8,), jnp.int32),
        pltpu.VMEM((BH,), jnp.int32),
        pltpu.VMEM((BH,), jnp.int32),
        pltpu.VMEM((BH,), jnp.int32),
        pltpu.VMEM((BH + 8,), jnp.int32),
        pltpu.VMEM((B, DIN), F32),
        pltpu.VMEM((B, DIN), F32),
        pltpu.SemaphoreType.DMA,
        pltpu.SemaphoreType.DMA,
        pltpu.SemaphoreType.DMA,
        pltpu.SemaphoreType.DMA,
        pltpu.SemaphoreType.DMA,
        pltpu.SemaphoreType.DMA,
    ],
)


BR = 2000


def _pre_body(xs, ws, bs, xd, wd, bd, os_, od_):
    os_[...] = jnp.dot(xs[...], ws[...], preferred_element_type=F32) + bs[...]
    od_[...] = jnp.dot(xd[...], wd[...], preferred_element_type=F32) + bd[...]


_pre_call = pl.pallas_call(
    _pre_body,
    grid=(N_NODE // BR,),
    in_specs=[
        pl.BlockSpec((BR, DIN), lambda i: (i, 0)),
        pl.BlockSpec((DIN, H * F), lambda i: (0, 0)),
        pl.BlockSpec((1, H * F), lambda i: (0, 0)),
        pl.BlockSpec((BR, DIN), lambda i: (i, 0)),
        pl.BlockSpec((DIN, H * F), lambda i: (0, 0)),
        pl.BlockSpec((1, H * F), lambda i: (0, 0)),
    ],
    out_specs=[pl.BlockSpec((BR, DIN), lambda i: (i, 0)),
               pl.BlockSpec((BR, DIN), lambda i: (i, 0))],
    out_shape=[jax.ShapeDtypeStruct((N_NODE, DIN), F32)] * 2,
)


def _post_body(s1, s2, dstf, hr, wih, bih, whh, bhh, wout, bout, kmat,
               out, hnew):
    S1 = s1[0] + s1[1]
    S2 = s2[0] + s2[1]
    den = jnp.dot(S2 + 1e-9, kmat[...], preferred_element_type=F32)
    x = jnp.maximum(S1 / den + dstf[...], 0.0)
    gi = jnp.dot(x, wih[...], preferred_element_type=F32) + bih[...]
    gh = jnp.dot(hr[...], whh[...], preferred_element_type=F32) + bhh[...]
    r = jax.nn.sigmoid(gi[:, :HID] + gh[:, :HID])
    z = jax.nn.sigmoid(gi[:, HID:2 * HID] + gh[:, HID:2 * HID])
    n = jnp.tanh(gi[:, 2 * HID:] + r * gh[:, 2 * HID:])
    hn = (1.0 - z) * n + z * hr[...]
    hnew[...] = hn
    out[...] = jnp.dot(hn, wout[...], preferred_element_type=F32) + bout[...]


_post_call = pl.pallas_call(
    _post_body,
    grid=(N_NODE // BR,),
    in_specs=[
        pl.BlockSpec((NC, BR, DIN), lambda i: (0, i, 0)),
        pl.BlockSpec((NC, BR, F), lambda i: (0, i, 0)),
        pl.BlockSpec((BR, DIN), lambda i: (i, 0)),
        pl.BlockSpec((BR, HID), lambda i: (i, 0)),
        pl.BlockSpec((HID, 3 * HID), lambda i: (0, 0)),
        pl.BlockSpec((1, 3 * HID), lambda i: (0, 0)),
        pl.BlockSpec((HID, 3 * HID), lambda i: (0, 0)),
        pl.BlockSpec((1, 3 * HID), lambda i: (0, 0)),
        pl.BlockSpec((HID, NACT), lambda i: (0, 0)),
        pl.BlockSpec((1, NACT), lambda i: (0, 0)),
        pl.BlockSpec((F, DIN), lambda i: (0, 0)),
    ],
    out_specs=[pl.BlockSpec((BR, NACT), lambda i: (i, 0)),
               pl.BlockSpec((BR, HID), lambda i: (i, 0))],
    out_shape=[jax.ShapeDtypeStruct((N_NODE, NACT), F32),
               jax.ShapeDtypeStruct((N_NODE, HID), F32)],
)


def kernel(src_feat, dst_feat, edge_index, h, W_src, b_src, W_dst, b_dst,
           attn, W_ih, b_ih, W_hh, b_hh, W_out, b_out):
    src1 = edge_index[0].astype(jnp.int32)
    dst1 = edge_index[1].astype(jnp.int32)
    d2 = dst1 // 8 + N_PAD
    dm8 = dst1 % 8

    feat_src, feat_dst = _pre_call(
        src_feat, W_src, b_src.reshape(1, -1),
        dst_feat, W_dst, b_dst.reshape(1, -1))

    acc = _edge_call(feat_src, feat_dst, src1, dst1, d2, dm8, attn)
    s1p = acc
    s2p = acc[:, N_PAD:, :].reshape(NC, N_PAD, F)

    kmat = (jnp.arange(DIN, dtype=jnp.int32)[None, :] // F
            == jnp.arange(F, dtype=jnp.int32)[:, None]).astype(F32)

    out, h_new = _post_call(
        s1p, s2p, dst_feat, h,
        W_ih.T, b_ih.reshape(1, -1), W_hh.T, b_hh.reshape(1, -1),
        W_out, b_out.reshape(1, -1), kmat)
    return (out, h_new)


# speedup vs baseline: 1.0323x; 1.0323x over previous
import jax
import jax.numpy as jnp
from jax import lax
from jax.experimental import pallas as pl
from jax.experimental.pallas import tpu as pltpu
from jax.experimental.pallas import tpu_sc as plsc

N_NODE = 10000
E_TOT = 320000
DIN = 128
H = 8
F = 16
HID = 128
NACT = 32

NC = 2
NS = 16
L = 16
NW = NC * NS
EPW = E_TOT // NW
B = 80
BH = 40
NHALF = EPW // BH
N_PAD = 10240
N_TOT = N_PAD + N_PAD // 8
RPT = N_TOT // NS

F32 = jnp.float32


def _edge_body(fs, fd, si, di, d2i, dmi, attn_h, out1,
               s_sh, attn_v,
               sidx0, didx0, d2b0, dmb0,
               sidx1, didx1, d2b1, dmb1,
               el, msg, sem_i0, sem_i1, sem_g0, sem_g1, sem_c0, sem_c1):
    c = lax.axis_index("c")
    s = lax.axis_index("s")
    wid = c * NS + s

    pltpu.sync_copy(attn_h, attn_v)

    def zrow(i, _):
        for j in range(DIN // L):
            msg[i, pl.ds(j * L, L)] = jnp.zeros((L,), F32)
            el[i, pl.ds(j * L, L)] = jnp.zeros((L,), F32)
        return 0
    lax.fori_loop(0, B, zrow, 0)
    for t in range(RPT // B):
        base = s * RPT + t * B
        pltpu.sync_copy(msg, s_sh.at[pl.ds(base, B)])
    plsc.subcore_barrier()

    lane = lax.broadcasted_iota(jnp.int32, (L,), 0)
    mask8 = lane < H
    perm = {d: lane ^ d for d in (1, 2, 4, 8)}
    mlow8 = lane < 8
    m4 = (lane & 4) == 0
    m2 = (lane & 2) == 0
    collect = (lane & 1) * 8 + (lane & 2) * 2 + (lane & 4) // 2
    zero16 = jnp.zeros((L,), F32)
    attn_regs = [attn_v[hh] for hh in range(H)]

    def _shuf(x, d):
        return x.at[perm[d]].get(mode="promise_in_bounds")

    def _dots8(ps):
        qs = [jnp.where(mlow8,
                        ps[2 * i] + _shuf(ps[2 * i], 8),
                        ps[2 * i + 1] + _shuf(ps[2 * i + 1], 8))
              for i in range(4)]
        rs = [jnp.where(m4,
                        qs[2 * i] + _shuf(qs[2 * i], 4),
                        qs[2 * i + 1] + _shuf(qs[2 * i + 1], 4))
              for i in range(2)]
        ss = jnp.where(m2, rs[0] + _shuf(rs[0], 2), rs[1] + _shuf(rs[1], 2))
        u = ss + _shuf(ss, 1)
        return u.at[collect].get(mode="promise_in_bounds")

    slot0 = (sidx0, didx0, d2b0, dmb0, sem_i0, sem_g0, sem_c0, 0)
    slot1 = (sidx1, didx1, d2b1, dmb1, sem_i1, sem_g1, sem_c1, BH)

    def issue_idx(h, st):
        sidx, didx, d2b, dmb, sem_i, _, _, _ = st
        off = pl.multiple_of(wid * EPW + h * BH, 8)
        pltpu.async_copy(si.at[pl.ds(off, BH)], sidx, sem_i)
        pltpu.async_copy(di.at[pl.ds(off, BH)], didx, sem_i)
        pltpu.async_copy(d2i.at[pl.ds(off, BH)], d2b, sem_i)
        pltpu.async_copy(dmi.at[pl.ds(off, BH)], dmb.at[pl.ds(0, BH)], sem_i)

    def wait_idx(st):
        sidx, didx, d2b, dmb, sem_i, _, _, _ = st
        pltpu.make_async_copy(si.at[pl.ds(0, BH)], sidx, sem_i).wait()
        pltpu.make_async_copy(di.at[pl.ds(0, BH)], didx, sem_i).wait()
        pltpu.make_async_copy(d2i.at[pl.ds(0, BH)], d2b, sem_i).wait()
        pltpu.make_async_copy(dmi.at[pl.ds(0, BH)], dmb.at[pl.ds(0, BH)],
                              sem_i).wait()

    def issue_gather(st):
        sidx, didx, _, _, _, sem_g, _, base = st
        pltpu.async_copy(fs.at[sidx], el.at[pl.ds(base, BH)], sem_g)
        pltpu.async_copy(fd.at[didx], msg.at[pl.ds(base, BH)], sem_g)

    def wait_gather(st):
        _, _, _, _, _, sem_g, _, base = st
        pltpu.make_async_copy(fs.at[pl.ds(0, BH)], el.at[pl.ds(base, BH)],
                              sem_g).wait()
        pltpu.make_async_copy(fd.at[pl.ds(0, BH)], msg.at[pl.ds(base, BH)],
                              sem_g).wait()

    def issue_scatter(st):
        _, didx, d2b, _, _, _, sem_c, base = st
        pltpu.async_copy(msg.at[pl.ds(base, BH)], s_sh.at[didx], sem_c,
                         add=True)
        pltpu.async_copy(el.at[pl.ds(base, BH)], s_sh.at[d2b], sem_c,
                         add=True)

    def wait_scatter(st):
        _, _, _, _, _, _, sem_c, _ = st
        pltpu.make_async_copy(fs.at[pl.ds(0, BH)], s_sh.at[pl.ds(0, BH)],
                              sem_c).wait()
        pltpu.make_async_copy(fs.at[pl.ds(0, BH)], s_sh.at[pl.ds(0, BH)],
                              sem_c).wait()

    def compute(st):
        _, _, _, dmb, _, _, _, base = st

        def edge(e, _):
            r = base + e
            evs = []
            ps = []
            for hh in range(H):
                a = el[r, pl.ds(hh * F, F)]
                b = msg[r, pl.ds(hh * F, F)]
                t = a + b
                t = jnp.maximum(t, 0.2 * t)
                ps.append(t * attn_regs[hh])
                evs.append(a)
            lgv = zero16
            for hh in range(H):
                v = ps[hh]
                for d in (1, 2, 4, 8):
                    v = v + _shuf(v, d)
                lgv = jnp.where(lane == hh, v, lgv)
            w = jnp.where(mask8, jnp.exp(lgv), 0.0)
            for hh in range(H):
                msg[r, pl.ds(hh * F, F)] = evs[hh] * w[hh]
            b16 = pl.multiple_of((e // L) * L, 8)
            dmv = dmb[pl.ds(b16, L)]
            dsel = dmv.at[lane * 0 + (e - b16)].get(mode="promise_in_bounds")
            for j in range(H):
                m = (1 - jnp.minimum(jnp.abs(dsel - j), 1)).astype(F32)
                el[r, pl.ds(j * F, F)] = w * m
            return 0
        lax.fori_loop(0, BH, edge, 0)

    def step(h, cur, nxt):
        _, didx, d2b, _, _, _, _, base = cur
        wait_gather(cur)
        wait_idx(nxt)
        issue_gather(nxt)
        compute(cur)
        pltpu.sync_copy(msg.at[pl.ds(base, BH)], s_sh.at[didx], add=True)
        pltpu.sync_copy(el.at[pl.ds(base, BH)], s_sh.at[d2b], add=True)
        issue_idx(lax.rem(h + 2, NHALF), cur)

    off0 = pl.multiple_of(wid * EPW, 8)
    pltpu.sync_copy(si.at[pl.ds(off0, BH)], sidx0)
    pltpu.sync_copy(di.at[pl.ds(off0, BH)], didx0)
    pltpu.sync_copy(d2i.at[pl.ds(off0, BH)], d2b0)
    pltpu.sync_copy(dmi.at[pl.ds(off0, BH)], dmb0.at[pl.ds(0, BH)])
    issue_gather(slot0)
    issue_idx(jnp.int32(1), slot1)

    def pair(kk, _):
        h0 = kk * 2
        step(h0, slot0, slot1)
        step(h0 + 1, slot1, slot0)
        return 0
    lax.fori_loop(0, NHALF // 2, pair, 0)

    wait_gather(slot0)
    wait_idx(slot1)

    plsc.subcore_barrier()
    base = s * RPT
    pltpu.sync_copy(s_sh.at[pl.ds(base, RPT)], out1.at[c, pl.ds(base, RPT)])


_edge_call = pl.kernel(
    _edge_body,
    out_type=jax.ShapeDtypeStruct((NC, N_TOT, DIN), F32),
    mesh=plsc.VectorSubcoreMesh(core_axis_name="c", subcore_axis_name="s",
                                num_cores=NC, num_subcores=NS),
    scratch_types=[
        pltpu.VMEM_SHARED((N_TOT, DIN), F32),
        pltpu.VMEM((H, F), F32),
        pltpu.VMEM((BH,), jnp.int32),
        pltpu.VMEM((BH,), jnp.int32),
        pltpu.VMEM((BH,), jnp.int32),
        pltpu.VMEM((BH + 8,), jnp.int32),
        pltpu.VMEM((BH,), jnp.int32),
        pltpu.VMEM((BH,), jnp.int32),
        pltpu.VMEM((BH,), jnp.int32),
        pltpu.VMEM((BH + 8,), jnp.int32),
        pltpu.VMEM((B, DIN), F32),
        pltpu.VMEM((B, DIN), F32),
        pltpu.SemaphoreType.DMA,
        pltpu.SemaphoreType.DMA,
        pltpu.SemaphoreType.DMA,
        pltpu.SemaphoreType.DMA,
        pltpu.SemaphoreType.DMA,
        pltpu.SemaphoreType.DMA,
    ],
)


BR = 2000


def _pre_body(xs, ws, bs, xd, wd, bd, os_, od_):
    os_[...] = jnp.dot(xs[...], ws[...], preferred_element_type=F32) + bs[...]
    od_[...] = jnp.dot(xd[...], wd[...], preferred_element_type=F32) + bd[...]


_pre_call = pl.pallas_call(
    _pre_body,
    grid=(N_NODE // BR,),
    in_specs=[
        pl.BlockSpec((BR, DIN), lambda i: (i, 0)),
        pl.BlockSpec((DIN, H * F), lambda i: (0, 0)),
        pl.BlockSpec((1, H * F), lambda i: (0, 0)),
        pl.BlockSpec((BR, DIN), lambda i: (i, 0)),
        pl.BlockSpec((DIN, H * F), lambda i: (0, 0)),
        pl.BlockSpec((1, H * F), lambda i: (0, 0)),
    ],
    out_specs=[pl.BlockSpec((BR, DIN), lambda i: (i, 0)),
               pl.BlockSpec((BR, DIN), lambda i: (i, 0))],
    out_shape=[jax.ShapeDtypeStruct((N_NODE, DIN), F32)] * 2,
)


def _post_body(s1, s2, dstf, hr, wih, bih, whh, bhh, wout, bout, kmat,
               out, hnew):
    S1 = s1[0] + s1[1]
    S2 = s2[0] + s2[1]
    den = jnp.dot(S2 + 1e-9, kmat[...], preferred_element_type=F32)
    x = jnp.maximum(S1 / den + dstf[...], 0.0)
    gi = jnp.dot(x, wih[...], preferred_element_type=F32) + bih[...]
    gh = jnp.dot(hr[...], whh[...], preferred_element_type=F32) + bhh[...]
    r = jax.nn.sigmoid(gi[:, :HID] + gh[:, :HID])
    z = jax.nn.sigmoid(gi[:, HID:2 * HID] + gh[:, HID:2 * HID])
    n = jnp.tanh(gi[:, 2 * HID:] + r * gh[:, 2 * HID:])
    hn = (1.0 - z) * n + z * hr[...]
    hnew[...] = hn
    out[...] = jnp.dot(hn, wout[...], preferred_element_type=F32) + bout[...]


_post_call = pl.pallas_call(
    _post_body,
    grid=(N_NODE // BR,),
    in_specs=[
        pl.BlockSpec((NC, BR, DIN), lambda i: (0, i, 0)),
        pl.BlockSpec((NC, BR, F), lambda i: (0, i, 0)),
        pl.BlockSpec((BR, DIN), lambda i: (i, 0)),
        pl.BlockSpec((BR, HID), lambda i: (i, 0)),
        pl.BlockSpec((HID, 3 * HID), lambda i: (0, 0)),
        pl.BlockSpec((1, 3 * HID), lambda i: (0, 0)),
        pl.BlockSpec((HID, 3 * HID), lambda i: (0, 0)),
        pl.BlockSpec((1, 3 * HID), lambda i: (0, 0)),
        pl.BlockSpec((HID, NACT), lambda i: (0, 0)),
        pl.BlockSpec((1, NACT), lambda i: (0, 0)),
        pl.BlockSpec((F, DIN), lambda i: (0, 0)),
    ],
    out_specs=[pl.BlockSpec((BR, NACT), lambda i: (i, 0)),
               pl.BlockSpec((BR, HID), lambda i: (i, 0))],
    out_shape=[jax.ShapeDtypeStruct((N_NODE, NACT), F32),
               jax.ShapeDtypeStruct((N_NODE, HID), F32)],
)


def kernel(src_feat, dst_feat, edge_index, h, W_src, b_src, W_dst, b_dst,
           attn, W_ih, b_ih, W_hh, b_hh, W_out, b_out):
    src1 = edge_index[0].astype(jnp.int32)
    dst1 = edge_index[1].astype(jnp.int32)
    d2 = dst1 // 8 + N_PAD
    dm8 = dst1 % 8

    feat_src, feat_dst = _pre_call(
        src_feat, W_src, b_src.reshape(1, -1),
        dst_feat, W_dst, b_dst.reshape(1, -1))

    acc = _edge_call(feat_src, feat_dst, src1, dst1, d2, dm8, attn)
    s1p = acc
    s2p = acc[:, N_PAD:, :].reshape(NC, N_PAD, F)

    kmat = (jnp.arange(DIN, dtype=jnp.int32)[None, :] // F
            == jnp.arange(F, dtype=jnp.int32)[:, None]).astype(F32)

    out, h_new = _post_call(
        s1p, s2p, dst_feat, h,
        W_ih.T, b_ih.reshape(1, -1), W_hh.T, b_hh.reshape(1, -1),
        W_out, b_out.reshape(1, -1), kmat)
    return (out, h_new)


# speedup vs baseline: 1.0448x; 1.0121x over previous
import jax
import jax.numpy as jnp
from jax import lax
from jax.experimental import pallas as pl
from jax.experimental.pallas import tpu as pltpu
from jax.experimental.pallas import tpu_sc as plsc

N_NODE = 10000
E_TOT = 320000
DIN = 128
H = 8
F = 16
HID = 128
NACT = 32

NC = 2
NS = 16
L = 16
NW = NC * NS
EPW = E_TOT // NW
B = 80
BH = 40
NHALF = EPW // BH
N_PAD = 10240
N_TOT = N_PAD + N_PAD // 8
RPT = N_TOT // NS

F32 = jnp.float32


def _edge_body(fs, fd, si, di, d2i, dmi, attn_h, out1,
               s_sh, attn_v,
               sidx0, didx0, d2b0, dmb0,
               sidx1, didx1, d2b1, dmb1,
               el, msg, sem_i0, sem_i1, sem_g0, sem_g1, sem_c0, sem_c1):
    c = lax.axis_index("c")
    s = lax.axis_index("s")
    wid = c * NS + s

    pltpu.sync_copy(attn_h, attn_v)

    def zrow(i, _):
        for j in range(DIN // L):
            msg[i, pl.ds(j * L, L)] = jnp.zeros((L,), F32)
            el[i, pl.ds(j * L, L)] = jnp.zeros((L,), F32)
        return 0
    lax.fori_loop(0, B, zrow, 0)
    for t in range(RPT // B):
        base = s * RPT + t * B
        pltpu.sync_copy(msg, s_sh.at[pl.ds(base, B)])
    plsc.subcore_barrier()

    lane = lax.broadcasted_iota(jnp.int32, (L,), 0)
    mask8 = lane < H
    perm = {d: lane ^ d for d in (1, 2, 4, 8)}
    mlow8 = lane < 8
    m4 = (lane & 4) == 0
    m2 = (lane & 2) == 0
    collect = (lane & 1) * 8 + (lane & 2) * 2 + (lane & 4) // 2
    zero16 = jnp.zeros((L,), F32)
    attn_regs = [attn_v[hh] for hh in range(H)]

    def _shuf(x, d):
        return x.at[perm[d]].get(mode="promise_in_bounds")

    def _dots8(ps):
        qs = [jnp.where(mlow8,
                        ps[2 * i] + _shuf(ps[2 * i], 8),
                        ps[2 * i + 1] + _shuf(ps[2 * i + 1], 8))
              for i in range(4)]
        rs = [jnp.where(m4,
                        qs[2 * i] + _shuf(qs[2 * i], 4),
                        qs[2 * i + 1] + _shuf(qs[2 * i + 1], 4))
              for i in range(2)]
        ss = jnp.where(m2, rs[0] + _shuf(rs[0], 2), rs[1] + _shuf(rs[1], 2))
        u = ss + _shuf(ss, 1)
        return u.at[collect].get(mode="promise_in_bounds")

    slot0 = (sidx0, didx0, d2b0, dmb0, sem_i0, sem_g0, sem_c0, 0)
    slot1 = (sidx1, didx1, d2b1, dmb1, sem_i1, sem_g1, sem_c1, BH)

    def issue_idx(h, st):
        sidx, didx, d2b, dmb, sem_i, _, _, _ = st
        off = pl.multiple_of(wid * EPW + h * BH, 8)
        pltpu.async_copy(si.at[pl.ds(off, BH)], sidx, sem_i)
        pltpu.async_copy(di.at[pl.ds(off, BH)], didx, sem_i)
        pltpu.async_copy(d2i.at[pl.ds(off, BH)], d2b, sem_i)
        pltpu.async_copy(dmi.at[pl.ds(off, BH)], dmb.at[pl.ds(0, BH)], sem_i)

    def wait_idx(st):
        sidx, didx, d2b, dmb, sem_i, _, _, _ = st
        pltpu.make_async_copy(si.at[pl.ds(0, BH)], sidx, sem_i).wait()
        pltpu.make_async_copy(di.at[pl.ds(0, BH)], didx, sem_i).wait()
        pltpu.make_async_copy(d2i.at[pl.ds(0, BH)], d2b, sem_i).wait()
        pltpu.make_async_copy(dmi.at[pl.ds(0, BH)], dmb.at[pl.ds(0, BH)],
                              sem_i).wait()

    def issue_gather(st):
        sidx, didx, _, _, _, sem_g, _, base = st
        pltpu.async_copy(fs.at[sidx], el.at[pl.ds(base, BH)], sem_g)
        pltpu.async_copy(fd.at[didx], msg.at[pl.ds(base, BH)], sem_g)

    def wait_gather(st):
        _, _, _, _, _, sem_g, _, base = st
        pltpu.make_async_copy(fs.at[pl.ds(0, BH)], el.at[pl.ds(base, BH)],
                              sem_g).wait()
        pltpu.make_async_copy(fd.at[pl.ds(0, BH)], msg.at[pl.ds(base, BH)],
                              sem_g).wait()

    def issue_scatter(st):
        _, didx, d2b, _, _, _, sem_c, base = st
        pltpu.async_copy(msg.at[pl.ds(base, BH)], s_sh.at[didx], sem_c,
                         add=True)
        pltpu.async_copy(el.at[pl.ds(base, BH)], s_sh.at[d2b], sem_c,
                         add=True)

    def wait_scatter(st):
        _, _, _, _, _, _, sem_c, _ = st
        pltpu.make_async_copy(fs.at[pl.ds(0, BH)], s_sh.at[pl.ds(0, BH)],
                              sem_c).wait()
        pltpu.make_async_copy(fs.at[pl.ds(0, BH)], s_sh.at[pl.ds(0, BH)],
                              sem_c).wait()

    def compute(st):
        _, _, _, dmb, _, _, _, base = st

        def edge(e, _):
            r = base + e
            evs = []
            ps = []
            for hh in range(H):
                a = el[r, pl.ds(hh * F, F)]
                b = msg[r, pl.ds(hh * F, F)]
                t = a + b
                t = jnp.maximum(t, 0.2 * t)
                ps.append(t * attn_regs[hh])
                evs.append(a)
            lgv = zero16
            for hh in range(H):
                v = ps[hh]
                for d in (1, 2, 4, 8):
                    v = v + _shuf(v, d)
                lgv = jnp.where(lane == hh, v, lgv)
            w = jnp.where(mask8, jnp.exp(lgv), 0.0)
            for hh in range(H):
                msg[r, pl.ds(hh * F, F)] = evs[hh] * w[hh]
            b16 = pl.multiple_of((e // L) * L, 8)
            dmv = dmb[pl.ds(b16, L)]
            dsel = dmv.at[lane * 0 + (e - b16)].get(mode="promise_in_bounds")
            for j in range(H):
                m = (1 - jnp.minimum(jnp.abs(dsel - j), 1)).astype(F32)
                el[r, pl.ds(j * F, F)] = w * m
            return 0
        lax.fori_loop(0, BH, edge, 0, unroll=2)

    def step(h, cur, nxt):
        _, didx, d2b, _, _, _, _, base = cur
        wait_gather(cur)
        wait_idx(nxt)
        issue_gather(nxt)
        compute(cur)
        pltpu.sync_copy(msg.at[pl.ds(base, BH)], s_sh.at[didx], add=True)
        pltpu.sync_copy(el.at[pl.ds(base, BH)], s_sh.at[d2b], add=True)
        issue_idx(lax.rem(h + 2, NHALF), cur)

    off0 = pl.multiple_of(wid * EPW, 8)
    pltpu.sync_copy(si.at[pl.ds(off0, BH)], sidx0)
    pltpu.sync_copy(di.at[pl.ds(off0, BH)], didx0)
    pltpu.sync_copy(d2i.at[pl.ds(off0, BH)], d2b0)
    pltpu.sync_copy(dmi.at[pl.ds(off0, BH)], dmb0.at[pl.ds(0, BH)])
    issue_gather(slot0)
    issue_idx(jnp.int32(1), slot1)

    def pair(kk, _):
        h0 = kk * 2
        step(h0, slot0, slot1)
        step(h0 + 1, slot1, slot0)
        return 0
    lax.fori_loop(0, NHALF // 2, pair, 0)

    wait_gather(slot0)
    wait_idx(slot1)

    plsc.subcore_barrier()
    base = s * RPT
    pltpu.sync_copy(s_sh.at[pl.ds(base, RPT)], out1.at[c, pl.ds(base, RPT)])


_edge_call = pl.kernel(
    _edge_body,
    out_type=jax.ShapeDtypeStruct((NC, N_TOT, DIN), F32),
    mesh=plsc.VectorSubcoreMesh(core_axis_name="c", subcore_axis_name="s",
                                num_cores=NC, num_subcores=NS),
    scratch_types=[
        pltpu.VMEM_SHARED((N_TOT, DIN), F32),
        pltpu.VMEM((H, F), F32),
        pltpu.VMEM((BH,), jnp.int32),
        pltpu.VMEM((BH,), jnp.int32),
        pltpu.VMEM((BH,), jnp.int32),
        pltpu.VMEM((BH + 8,), jnp.int32),
        pltpu.VMEM((BH,), jnp.int32),
        pltpu.VMEM((BH,), jnp.int32),
        pltpu.VMEM((BH,), jnp.int32),
        pltpu.VMEM((BH + 8,), jnp.int32),
        pltpu.VMEM((B, DIN), F32),
        pltpu.VMEM((B, DIN), F32),
        pltpu.SemaphoreType.DMA,
        pltpu.SemaphoreType.DMA,
        pltpu.SemaphoreType.DMA,
        pltpu.SemaphoreType.DMA,
        pltpu.SemaphoreType.DMA,
        pltpu.SemaphoreType.DMA,
    ],
)


BR = 2000


def _pre_body(xs, ws, bs, xd, wd, bd, os_, od_):
    os_[...] = jnp.dot(xs[...], ws[...], preferred_element_type=F32) + bs[...]
    od_[...] = jnp.dot(xd[...], wd[...], preferred_element_type=F32) + bd[...]


_pre_call = pl.pallas_call(
    _pre_body,
    grid=(N_NODE // BR,),
    in_specs=[
        pl.BlockSpec((BR, DIN), lambda i: (i, 0)),
        pl.BlockSpec((DIN, H * F), lambda i: (0, 0)),
        pl.BlockSpec((1, H * F), lambda i: (0, 0)),
        pl.BlockSpec((BR, DIN), lambda i: (i, 0)),
        pl.BlockSpec((DIN, H * F), lambda i: (0, 0)),
        pl.BlockSpec((1, H * F), lambda i: (0, 0)),
    ],
    out_specs=[pl.BlockSpec((BR, DIN), lambda i: (i, 0)),
               pl.BlockSpec((BR, DIN), lambda i: (i, 0))],
    out_shape=[jax.ShapeDtypeStruct((N_NODE, DIN), F32)] * 2,
)


def _post_body(s1, s2, dstf, hr, wih, bih, whh, bhh, wout, bout, kmat,
               out, hnew):
    S1 = s1[0] + s1[1]
    S2 = s2[0] + s2[1]
    den = jnp.dot(S2 + 1e-9, kmat[...], preferred_element_type=F32)
    x = jnp.maximum(S1 / den + dstf[...], 0.0)
    gi = jnp.dot(x, wih[...], preferred_element_type=F32) + bih[...]
    gh = jnp.dot(hr[...], whh[...], preferred_element_type=F32) + bhh[...]
    r = jax.nn.sigmoid(gi[:, :HID] + gh[:, :HID])
    z = jax.nn.sigmoid(gi[:, HID:2 * HID] + gh[:, HID:2 * HID])
    n = jnp.tanh(gi[:, 2 * HID:] + r * gh[:, 2 * HID:])
    hn = (1.0 - z) * n + z * hr[...]
    hnew[...] = hn
    out[...] = jnp.dot(hn, wout[...], preferred_element_type=F32) + bout[...]


_post_call = pl.pallas_call(
    _post_body,
    grid=(N_NODE // BR,),
    in_specs=[
        pl.BlockSpec((NC, BR, DIN), lambda i: (0, i, 0)),
        pl.BlockSpec((NC, BR, F), lambda i: (0, i, 0)),
        pl.BlockSpec((BR, DIN), lambda i: (i, 0)),
        pl.BlockSpec((BR, HID), lambda i: (i, 0)),
        pl.BlockSpec((HID, 3 * HID), lambda i: (0, 0)),
        pl.BlockSpec((1, 3 * HID), lambda i: (0, 0)),
        pl.BlockSpec((HID, 3 * HID), lambda i: (0, 0)),
        pl.BlockSpec((1, 3 * HID), lambda i: (0, 0)),
        pl.BlockSpec((HID, NACT), lambda i: (0, 0)),
        pl.BlockSpec((1, NACT), lambda i: (0, 0)),
        pl.BlockSpec((F, DIN), lambda i: (0, 0)),
    ],
    out_specs=[pl.BlockSpec((BR, NACT), lambda i: (i, 0)),
               pl.BlockSpec((BR, HID), lambda i: (i, 0))],
    out_shape=[jax.ShapeDtypeStruct((N_NODE, NACT), F32),
               jax.ShapeDtypeStruct((N_NODE, HID), F32)],
)


def kernel(src_feat, dst_feat, edge_index, h, W_src, b_src, W_dst, b_dst,
           attn, W_ih, b_ih, W_hh, b_hh, W_out, b_out):
    src1 = edge_index[0].astype(jnp.int32)
    dst1 = edge_index[1].astype(jnp.int32)
    d2 = dst1 // 8 + N_PAD
    dm8 = dst1 % 8

    feat_src, feat_dst = _pre_call(
        src_feat, W_src, b_src.reshape(1, -1),
        dst_feat, W_dst, b_dst.reshape(1, -1))

    acc = _edge_call(feat_src, feat_dst, src1, dst1, d2, dm8, attn)
    s1p = acc
    s2p = acc[:, N_PAD:, :].reshape(NC, N_PAD, F)

    kmat = (jnp.arange(DIN, dtype=jnp.int32)[None, :] // F
            == jnp.arange(F, dtype=jnp.int32)[:, None]).astype(F32)

    out, h_new = _post_call(
        s1p, s2p, dst_feat, h,
        W_ih.T, b_ih.reshape(1, -1), W_hh.T, b_hh.reshape(1, -1),
        W_out, b_out.reshape(1, -1), kmat)
    return (out, h_new)


# speedup vs baseline: 1.0851x; 1.0386x over previous
import jax
import jax.numpy as jnp
from jax import lax
from jax.experimental import pallas as pl
from jax.experimental.pallas import tpu as pltpu
from jax.experimental.pallas import tpu_sc as plsc

N_NODE = 10000
E_TOT = 320000
DIN = 128
H = 8
F = 16
HID = 128
NACT = 32

NC = 2
NS = 16
L = 16
NW = NC * NS
EPW = E_TOT // NW
B = 80
BH = 40
NHALF = EPW // BH
N_PAD = 10240
N_TOT = N_PAD + N_PAD // 8
RPT = N_TOT // NS

F32 = jnp.float32


def _edge_body(fs, fd, si, di, d2i, dmi, attn_h, out1,
               s_sh, attn_v,
               sidx0, didx0, d2b0, dmb0,
               sidx1, didx1, d2b1, dmb1,
               el, msg, sem_i0, sem_i1, sem_g0, sem_g1, sem_c0, sem_c1):
    c = lax.axis_index("c")
    s = lax.axis_index("s")
    wid = c * NS + s

    pltpu.sync_copy(attn_h, attn_v)

    def zrow(i, _):
        for j in range(DIN // L):
            msg[i, pl.ds(j * L, L)] = jnp.zeros((L,), F32)
            el[i, pl.ds(j * L, L)] = jnp.zeros((L,), F32)
        return 0
    lax.fori_loop(0, B, zrow, 0)
    for t in range(RPT // B):
        base = s * RPT + t * B
        pltpu.sync_copy(msg, s_sh.at[pl.ds(base, B)])
    plsc.subcore_barrier()

    lane = lax.broadcasted_iota(jnp.int32, (L,), 0)
    mask8 = lane < H
    perm = {d: lane ^ d for d in (1, 2, 4, 8)}
    mlow8 = lane < 8
    m4 = (lane & 4) == 0
    m2 = (lane & 2) == 0
    collect = (lane & 1) * 8 + (lane & 2) * 2 + (lane & 4) // 2
    zero16 = jnp.zeros((L,), F32)
    attn_regs = [attn_v[hh] for hh in range(H)]

    def _shuf(x, d):
        return x.at[perm[d]].get(mode="promise_in_bounds")

    def _dots8(ps):
        qs = [jnp.where(mlow8,
                        ps[2 * i] + _shuf(ps[2 * i], 8),
                        ps[2 * i + 1] + _shuf(ps[2 * i + 1], 8))
              for i in range(4)]
        rs = [jnp.where(m4,
                        qs[2 * i] + _shuf(qs[2 * i], 4),
                        qs[2 * i + 1] + _shuf(qs[2 * i + 1], 4))
              for i in range(2)]
        ss = jnp.where(m2, rs[0] + _shuf(rs[0], 2), rs[1] + _shuf(rs[1], 2))
        u = ss + _shuf(ss, 1)
        return u.at[collect].get(mode="promise_in_bounds")

    slot0 = (sidx0, didx0, d2b0, dmb0, sem_i0, sem_g0, sem_c0, 0)
    slot1 = (sidx1, didx1, d2b1, dmb1, sem_i1, sem_g1, sem_c1, BH)

    def issue_idx(h, st):
        sidx, didx, d2b, dmb, sem_i, _, _, _ = st
        off = pl.multiple_of(wid * EPW + h * BH, 8)
        pltpu.async_copy(si.at[pl.ds(off, BH)], sidx, sem_i)
        pltpu.async_copy(di.at[pl.ds(off, BH)], didx, sem_i)
        pltpu.async_copy(d2i.at[pl.ds(off, BH)], d2b, sem_i)
        pltpu.async_copy(dmi.at[pl.ds(off, BH)], dmb.at[pl.ds(0, BH)], sem_i)

    def wait_idx(st):
        sidx, didx, d2b, dmb, sem_i, _, _, _ = st
        pltpu.make_async_copy(si.at[pl.ds(0, BH)], sidx, sem_i).wait()
        pltpu.make_async_copy(di.at[pl.ds(0, BH)], didx, sem_i).wait()
        pltpu.make_async_copy(d2i.at[pl.ds(0, BH)], d2b, sem_i).wait()
        pltpu.make_async_copy(dmi.at[pl.ds(0, BH)], dmb.at[pl.ds(0, BH)],
                              sem_i).wait()

    def issue_gather(st):
        sidx, didx, _, _, _, sem_g, _, base = st
        pltpu.async_copy(fs.at[sidx], el.at[pl.ds(base, BH)], sem_g)
        pltpu.async_copy(fd.at[didx], msg.at[pl.ds(base, BH)], sem_g)

    def wait_gather(st):
        _, _, _, _, _, sem_g, _, base = st
        pltpu.make_async_copy(fs.at[pl.ds(0, BH)], el.at[pl.ds(base, BH)],
                              sem_g).wait()
        pltpu.make_async_copy(fd.at[pl.ds(0, BH)], msg.at[pl.ds(base, BH)],
                              sem_g).wait()

    def issue_scatter(st):
        _, didx, d2b, _, _, _, sem_c, base = st
        pltpu.async_copy(msg.at[pl.ds(base, BH)], s_sh.at[didx], sem_c,
                         add=True)
        pltpu.async_copy(el.at[pl.ds(base, BH)], s_sh.at[d2b], sem_c,
                         add=True)

    def wait_scatter(st):
        _, _, _, _, _, _, sem_c, _ = st
        pltpu.make_async_copy(fs.at[pl.ds(0, BH)], s_sh.at[pl.ds(0, BH)],
                              sem_c).wait()
        pltpu.make_async_copy(fs.at[pl.ds(0, BH)], s_sh.at[pl.ds(0, BH)],
                              sem_c).wait()

    def compute(st):
        _, _, _, dmb, _, _, _, base = st

        def edge(e, _):
            r = base + e
            evs = []
            ps = []
            for hh in range(H):
                a = el[r, pl.ds(hh * F, F)]
                b = msg[r, pl.ds(hh * F, F)]
                t = a + b
                t = jnp.maximum(t, 0.2 * t)
                ps.append(t * attn_regs[hh])
                evs.append(a)
            lgv = zero16
            for hh in range(H):
                v = ps[hh]
                for d in (1, 2, 4, 8):
                    v = v + _shuf(v, d)
                lgv = jnp.where(lane == hh, v, lgv)
            w = jnp.where(mask8, jnp.exp(lgv), 0.0)
            for hh in range(H):
                msg[r, pl.ds(hh * F, F)] = evs[hh] * w[hh]
            b16 = pl.multiple_of((e // L) * L, 8)
            dmv = dmb[pl.ds(b16, L)]
            dsel = dmv.at[lane * 0 + (e - b16)].get(mode="promise_in_bounds")
            dself = dsel.astype(F32)
            for j in range(H):
                m = jnp.maximum(0.0, 1.0 - jnp.abs(dself - float(j)))
                el[r, pl.ds(j * F, F)] = w * m
            return 0
        lax.fori_loop(0, BH, edge, 0, unroll=2)

    def step(h, cur, nxt):
        _, didx, d2b, _, _, _, _, base = cur
        wait_gather(cur)
        wait_idx(nxt)
        issue_gather(nxt)
        compute(cur)
        pltpu.sync_copy(msg.at[pl.ds(base, BH)], s_sh.at[didx], add=True)
        pltpu.sync_copy(el.at[pl.ds(base, BH)], s_sh.at[d2b], add=True)
        issue_idx(lax.rem(h + 2, NHALF), cur)

    off0 = pl.multiple_of(wid * EPW, 8)
    pltpu.sync_copy(si.at[pl.ds(off0, BH)], sidx0)
    pltpu.sync_copy(di.at[pl.ds(off0, BH)], didx0)
    pltpu.sync_copy(d2i.at[pl.ds(off0, BH)], d2b0)
    pltpu.sync_copy(dmi.at[pl.ds(off0, BH)], dmb0.at[pl.ds(0, BH)])
    issue_gather(slot0)
    issue_idx(jnp.int32(1), slot1)

    def pair(kk, _):
        h0 = kk * 2
        step(h0, slot0, slot1)
        step(h0 + 1, slot1, slot0)
        return 0
    lax.fori_loop(0, NHALF // 2, pair, 0)

    wait_gather(slot0)
    wait_idx(slot1)

    plsc.subcore_barrier()
    base = s * RPT
    pltpu.sync_copy(s_sh.at[pl.ds(base, RPT)], out1.at[c, pl.ds(base, RPT)])


_edge_call = pl.kernel(
    _edge_body,
    out_type=jax.ShapeDtypeStruct((NC, N_TOT, DIN), F32),
    mesh=plsc.VectorSubcoreMesh(core_axis_name="c", subcore_axis_name="s",
                                num_cores=NC, num_subcores=NS),
    scratch_types=[
        pltpu.VMEM_SHARED((N_TOT, DIN), F32),
        pltpu.VMEM((H, F), F32),
        pltpu.VMEM((BH,), jnp.int32),
        pltpu.VMEM((BH,), jnp.int32),
        pltpu.VMEM((BH,), jnp.int32),
        pltpu.VMEM((BH + 8,), jnp.int32),
        pltpu.VMEM((BH,), jnp.int32),
        pltpu.VMEM((BH,), jnp.int32),
        pltpu.VMEM((BH,), jnp.int32),
        pltpu.VMEM((BH + 8,), jnp.int32),
        pltpu.VMEM((B, DIN), F32),
        pltpu.VMEM((B, DIN), F32),
        pltpu.SemaphoreType.DMA,
        pltpu.SemaphoreType.DMA,
        pltpu.SemaphoreType.DMA,
        pltpu.SemaphoreType.DMA,
        pltpu.SemaphoreType.DMA,
        pltpu.SemaphoreType.DMA,
    ],
)


BR = 2000


def _pre_body(xs, ws, bs, xd, wd, bd, os_, od_):
    os_[...] = jnp.dot(xs[...], ws[...], preferred_element_type=F32) + bs[...]
    od_[...] = jnp.dot(xd[...], wd[...], preferred_element_type=F32) + bd[...]


_pre_call = pl.pallas_call(
    _pre_body,
    grid=(N_NODE // BR,),
    in_specs=[
        pl.BlockSpec((BR, DIN), lambda i: (i, 0)),
        pl.BlockSpec((DIN, H * F), lambda i: (0, 0)),
        pl.BlockSpec((1, H * F), lambda i: (0, 0)),
        pl.BlockSpec((BR, DIN), lambda i: (i, 0)),
        pl.BlockSpec((DIN, H * F), lambda i: (0, 0)),
        pl.BlockSpec((1, H * F), lambda i: (0, 0)),
    ],
    out_specs=[pl.BlockSpec((BR, DIN), lambda i: (i, 0)),
               pl.BlockSpec((BR, DIN), lambda i: (i, 0))],
    out_shape=[jax.ShapeDtypeStruct((N_NODE, DIN), F32)] * 2,
)


def _post_body(s1, s2, dstf, hr, wih, bih, whh, bhh, wout, bout, kmat,
               out, hnew):
    S1 = s1[0] + s1[1]
    S2 = s2[0] + s2[1]
    den = jnp.dot(S2 + 1e-9, kmat[...], preferred_element_type=F32)
    x = jnp.maximum(S1 / den + dstf[...], 0.0)
    gi = jnp.dot(x, wih[...], preferred_element_type=F32) + bih[...]
    gh = jnp.dot(hr[...], whh[...], preferred_element_type=F32) + bhh[...]
    r = jax.nn.sigmoid(gi[:, :HID] + gh[:, :HID])
    z = jax.nn.sigmoid(gi[:, HID:2 * HID] + gh[:, HID:2 * HID])
    n = jnp.tanh(gi[:, 2 * HID:] + r * gh[:, 2 * HID:])
    hn = (1.0 - z) * n + z * hr[...]
    hnew[...] = hn
    out[...] = jnp.dot(hn, wout[...], preferred_element_type=F32) + bout[...]


_post_call = pl.pallas_call(
    _post_body,
    grid=(N_NODE // BR,),
    in_specs=[
        pl.BlockSpec((NC, BR, DIN), lambda i: (0, i, 0)),
        pl.BlockSpec((NC, BR, F), lambda i: (0, i, 0)),
        pl.BlockSpec((BR, DIN), lambda i: (i, 0)),
        pl.BlockSpec((BR, HID), lambda i: (i, 0)),
        pl.BlockSpec((HID, 3 * HID), lambda i: (0, 0)),
        pl.BlockSpec((1, 3 * HID), lambda i: (0, 0)),
        pl.BlockSpec((HID, 3 * HID), lambda i: (0, 0)),
        pl.BlockSpec((1, 3 * HID), lambda i: (0, 0)),
        pl.BlockSpec((HID, NACT), lambda i: (0, 0)),
        pl.BlockSpec((1, NACT), lambda i: (0, 0)),
        pl.BlockSpec((F, DIN), lambda i: (0, 0)),
    ],
    out_specs=[pl.BlockSpec((BR, NACT), lambda i: (i, 0)),
               pl.BlockSpec((BR, HID), lambda i: (i, 0))],
    out_shape=[jax.ShapeDtypeStruct((N_NODE, NACT), F32),
               jax.ShapeDtypeStruct((N_NODE, HID), F32)],
)


def kernel(src_feat, dst_feat, edge_index, h, W_src, b_src, W_dst, b_dst,
           attn, W_ih, b_ih, W_hh, b_hh, W_out, b_out):
    src1 = edge_index[0].astype(jnp.int32)
    dst1 = edge_index[1].astype(jnp.int32)
    d2 = dst1 // 8 + N_PAD
    dm8 = dst1 % 8

    feat_src, feat_dst = _pre_call(
        src_feat, W_src, b_src.reshape(1, -1),
        dst_feat, W_dst, b_dst.reshape(1, -1))

    acc = _edge_call(feat_src, feat_dst, src1, dst1, d2, dm8, attn)
    s1p = acc
    s2p = acc[:, N_PAD:, :].reshape(NC, N_PAD, F)

    kmat = (jnp.arange(DIN, dtype=jnp.int32)[None, :] // F
            == jnp.arange(F, dtype=jnp.int32)[:, None]).astype(F32)

    out, h_new = _post_call(
        s1p, s2p, dst_feat, h,
        W_ih.T, b_ih.reshape(1, -1), W_hh.T, b_hh.reshape(1, -1),
        W_out, b_out.reshape(1, -1), kmat)
    return (out, h_new)
